# baseline (speedup 1.0000x reference)
import jax
import jax.numpy as jnp
from jax import lax
from jax.experimental import pallas as pl
from jax.experimental.pallas import tpu as pltpu
from jax.experimental.pallas import tpu_sc as plsc

EDIM = 64
BATCH = 16384

_NC = 2
_NS = 16
_NW = _NC * _NS
_BPW = BATCH // _NW
_CHUNK = 128
_NCHUNK = _BPW // _CHUNK

_CB = 2048


def _repack_body(ta_ref, tb_ref, out_ref):
  eye = (lax.broadcasted_iota(jnp.int32, (EDIM, EDIM), 0) ==
         lax.broadcasted_iota(jnp.int32, (EDIM, EDIM), 1)).astype(jnp.float32)
  dn = (((0,), (0,)), ((), ()))
  out_ref[:, :EDIM] = lax.dot_general(
      ta_ref[...], eye, dn, preferred_element_type=jnp.float32)
  out_ref[:, EDIM:] = lax.dot_general(
      tb_ref[...], eye, dn, preferred_element_type=jnp.float32)


def _repack(table_t, half):
  n = table_t.shape[1]
  steps = half // _CB
  last = (n + _CB - 1) // _CB - 1
  return pl.pallas_call(
      _repack_body,
      grid=(steps,),
      in_specs=[
          pl.BlockSpec((EDIM, _CB), lambda i: (0, i)),
          pl.BlockSpec((EDIM, _CB),
                       lambda i: (0, jnp.minimum(i + steps, last))),
      ],
      out_specs=pl.BlockSpec((_CB, 2 * EDIM), lambda i: (i, 0)),
      out_shape=jax.ShapeDtypeStruct((half, 2 * EDIM), jnp.float32),
  )(table_t, table_t)


def _sc_gather_body(x2_hbm, x3_hbm, e2_hbm, e3_hbm, out2_hbm, out3_hbm,
                    idx_v, rows_v, sem):
  wid = lax.axis_index("s") * _NC + lax.axis_index("c")
  base = wid * _BPW
  pltpu.sync_copy(x2_hbm.at[pl.ds(wid * _NCHUNK, _NCHUNK)], idx_v.at[0])
  pltpu.sync_copy(x3_hbm.at[pl.ds(wid * _NCHUNK, _NCHUNK)], idx_v.at[1])
  for t, (e_hbm, out_hbm) in enumerate(((e2_hbm, out2_hbm),
                                        (e3_hbm, out3_hbm))):
    copies = [
        pltpu.async_copy(e_hbm.at[idx_v.at[t].at[c]],
                         rows_v.at[pl.ds(c * _CHUNK, _CHUNK)], sem)
        for c in range(_NCHUNK)
    ]
    for cp in copies:
      cp.wait()
    pltpu.sync_copy(rows_v, out_hbm.at[pl.ds(base, _BPW)])


def _sc_gather(x2h, x3h, e2v, e3v):
  mesh = plsc.VectorSubcoreMesh(core_axis_name="c", subcore_axis_name="s")
  f = pl.kernel(
      _sc_gather_body,
      mesh=mesh,
      out_type=(
          jax.ShapeDtypeStruct((BATCH, 2 * EDIM), jnp.float32),
          jax.ShapeDtypeStruct((BATCH, 2 * EDIM), jnp.float32),
      ),
      scratch_types=[
          pltpu.VMEM((2, _NCHUNK, _CHUNK), jnp.int32),
          pltpu.VMEM((_BPW, 2 * EDIM), jnp.float32),
          pltpu.SemaphoreType.DMA,
      ],
  )
  return f(x2h.reshape(_NW * _NCHUNK, _CHUNK),
           x3h.reshape(_NW * _NCHUNK, _CHUNK), e2v, e3v)


_BM = 2048


def _mlp_body(g2_ref, g3_ref, p2_ref, p3_ref, w1a_ref, w1b_ref, b1_ref,
              w2_ref, b2_ref, w3_ref, b3_ref, w4_ref, b4_ref, out_ref):
  g2 = g2_ref[...]
  g3 = g3_ref[...]
  emb2 = jnp.where(p2_ref[...] > 0, g2[:, EDIM:], g2[:, :EDIM])
  emb3 = jnp.where(p3_ref[...] > 0, g3[:, EDIM:], g3[:, :EDIM])
  h = jnp.dot(emb2, w1a_ref[...], preferred_element_type=jnp.float32)
  h = h + jnp.dot(emb3, w1b_ref[...], preferred_element_type=jnp.float32)
  h = jnp.maximum(h + b1_ref[...], 0.0)
  h = jnp.maximum(
      jnp.dot(h, w2_ref[...], preferred_element_type=jnp.float32) + b2_ref[...],
      0.0)
  h = jnp.maximum(
      jnp.dot(h, w3_ref[...], preferred_element_type=jnp.float32) + b3_ref[...],
      0.0)
  out_ref[...] = (
      jnp.dot(h, w4_ref[...], preferred_element_type=jnp.float32) + b4_ref[...])


def _mlp(g2, g3, p2, p3, W1, b1, W2, b2, W3, b3, W4, b4):
  w1a = W1[:EDIM]
  w1b = W1[EDIM:]
  full = lambda i: (0, 0)
  return pl.pallas_call(
      _mlp_body,
      grid=(BATCH // _BM,),
      in_specs=[
          pl.BlockSpec((_BM, 2 * EDIM), lambda i: (i, 0)),
          pl.BlockSpec((_BM, 2 * EDIM), lambda i: (i, 0)),
          pl.BlockSpec((_BM, 1), lambda i: (i, 0)),
          pl.BlockSpec((_BM, 1), lambda i: (i, 0)),
          pl.BlockSpec(w1a.shape, full),
          pl.BlockSpec(w1b.shape, full),
          pl.BlockSpec((1, 32), full),
          pl.BlockSpec(W2.shape, full),
          pl.BlockSpec((1, 32), full),
          pl.BlockSpec(W3.shape, full),
          pl.BlockSpec((1, 16), full),
          pl.BlockSpec(W4.shape, full),
          pl.BlockSpec((1, 3), full),
      ],
      out_specs=pl.BlockSpec((_BM, 3), lambda i: (i, 0)),
      out_shape=jax.ShapeDtypeStruct((BATCH, 3), jnp.float32),
  )(g2, g3, p2.reshape(BATCH, 1), p3.reshape(BATCH, 1), w1a, w1b,
    b1.reshape(1, 32), W2, b2.reshape(1, 32), W3, b3.reshape(1, 16), W4,
    b4.reshape(1, 3))


def kernel(X_2, X_3, E2, E3, W1, b1, W2, b2, W3, b3, W4, b4):
  x2 = X_2.astype(jnp.int32)
  x3 = X_3.astype(jnp.int32)
  h2 = _CB * ((E2.shape[0] // 2 + _CB - 1) // _CB)
  h3 = _CB * ((E3.shape[0] // 2 + _CB - 1) // _CB)
  e2v = _repack(jnp.transpose(E2), h2)
  e3v = _repack(jnp.transpose(E3), h3)
  s2 = (x2 >= h2).astype(jnp.int32)
  s3 = (x3 >= h3).astype(jnp.int32)
  g2, g3 = _sc_gather(x2 - s2 * h2, x3 - s3 * h3, e2v, e3v)
  return _mlp(g2, g3, s2, s3, W1, b1, W2, b2, W3, b3, W4, b4)


# speedup vs baseline: 1.3333x; 1.3333x over previous
import jax
import jax.numpy as jnp
from jax import lax
from jax.experimental import pallas as pl
from jax.experimental.pallas import tpu as pltpu
from jax.experimental.pallas import tpu_sc as plsc

EDIM = 64
BATCH = 16384

_NC = 2
_NS = 16
_NW = _NC * _NS
_BPW = BATCH // _NW
_CHUNK = 128
_NCHUNK = _BPW // _CHUNK

_CB = 8192


def _repack_body(ta_ref, tb_ref, out_ref):
  eye = (lax.broadcasted_iota(jnp.int32, (EDIM, EDIM), 0) ==
         lax.broadcasted_iota(jnp.int32, (EDIM, EDIM), 1)).astype(jnp.float32)
  dn = (((0,), (0,)), ((), ()))
  out_ref[:, :EDIM] = lax.dot_general(
      ta_ref[...], eye, dn, preferred_element_type=jnp.float32)
  out_ref[:, EDIM:] = lax.dot_general(
      tb_ref[...], eye, dn, preferred_element_type=jnp.float32)


def _repack(table_t, half):
  n = table_t.shape[1]
  steps = half // _CB
  last = (n + _CB - 1) // _CB - 1
  return pl.pallas_call(
      _repack_body,
      grid=(steps,),
      in_specs=[
          pl.BlockSpec((EDIM, _CB), lambda i: (0, i)),
          pl.BlockSpec((EDIM, _CB),
                       lambda i: (0, jnp.minimum(i + steps, last))),
      ],
      out_specs=pl.BlockSpec((_CB, 2 * EDIM), lambda i: (i, 0)),
      out_shape=jax.ShapeDtypeStruct((half, 2 * EDIM), jnp.float32),
  )(table_t, table_t)


def _sc_gather_body(x2_hbm, x3_hbm, e2_hbm, e3_hbm, out2_hbm, out3_hbm,
                    idx_v, rows_v, sem):
  wid = lax.axis_index("s") * _NC + lax.axis_index("c")
  base = wid * _BPW
  pltpu.sync_copy(x2_hbm.at[pl.ds(wid * _NCHUNK, _NCHUNK)], idx_v.at[0])
  pltpu.sync_copy(x3_hbm.at[pl.ds(wid * _NCHUNK, _NCHUNK)], idx_v.at[1])
  for t, (e_hbm, out_hbm) in enumerate(((e2_hbm, out2_hbm),
                                        (e3_hbm, out3_hbm))):
    copies = [
        pltpu.async_copy(e_hbm.at[idx_v.at[t].at[c]],
                         rows_v.at[pl.ds(c * _CHUNK, _CHUNK)], sem)
        for c in range(_NCHUNK)
    ]
    for cp in copies:
      cp.wait()
    pltpu.sync_copy(rows_v, out_hbm.at[pl.ds(base, _BPW)])


def _sc_gather(x2h, x3h, e2v, e3v):
  mesh = plsc.VectorSubcoreMesh(core_axis_name="c", subcore_axis_name="s")
  f = pl.kernel(
      _sc_gather_body,
      mesh=mesh,
      out_type=(
          jax.ShapeDtypeStruct((BATCH, 2 * EDIM), jnp.float32),
          jax.ShapeDtypeStruct((BATCH, 2 * EDIM), jnp.float32),
      ),
      scratch_types=[
          pltpu.VMEM((2, _NCHUNK, _CHUNK), jnp.int32),
          pltpu.VMEM((_BPW, 2 * EDIM), jnp.float32),
          pltpu.SemaphoreType.DMA,
      ],
  )
  return f(x2h.reshape(_NW * _NCHUNK, _CHUNK),
           x3h.reshape(_NW * _NCHUNK, _CHUNK), e2v, e3v)


_BM = 2048


def _mlp_body(g2_ref, g3_ref, p2_ref, p3_ref, w1a_ref, w1b_ref, b1_ref,
              w2_ref, b2_ref, w3_ref, b3_ref, w4_ref, b4_ref, out_ref):
  g2 = g2_ref[...]
  g3 = g3_ref[...]
  emb2 = jnp.where(p2_ref[...] > 0, g2[:, EDIM:], g2[:, :EDIM])
  emb3 = jnp.where(p3_ref[...] > 0, g3[:, EDIM:], g3[:, :EDIM])
  h = jnp.dot(emb2, w1a_ref[...], preferred_element_type=jnp.float32)
  h = h + jnp.dot(emb3, w1b_ref[...], preferred_element_type=jnp.float32)
  h = jnp.maximum(h + b1_ref[...], 0.0)
  h = jnp.maximum(
      jnp.dot(h, w2_ref[...], preferred_element_type=jnp.float32) + b2_ref[...],
      0.0)
  h = jnp.maximum(
      jnp.dot(h, w3_ref[...], preferred_element_type=jnp.float32) + b3_ref[...],
      0.0)
  out_ref[...] = (
      jnp.dot(h, w4_ref[...], preferred_element_type=jnp.float32) + b4_ref[...])


def _mlp(g2, g3, p2, p3, W1, b1, W2, b2, W3, b3, W4, b4):
  w1a = W1[:EDIM]
  w1b = W1[EDIM:]
  full = lambda i: (0, 0)
  return pl.pallas_call(
      _mlp_body,
      grid=(BATCH // _BM,),
      in_specs=[
          pl.BlockSpec((_BM, 2 * EDIM), lambda i: (i, 0)),
          pl.BlockSpec((_BM, 2 * EDIM), lambda i: (i, 0)),
          pl.BlockSpec((_BM, 1), lambda i: (i, 0)),
          pl.BlockSpec((_BM, 1), lambda i: (i, 0)),
          pl.BlockSpec(w1a.shape, full),
          pl.BlockSpec(w1b.shape, full),
          pl.BlockSpec((1, 32), full),
          pl.BlockSpec(W2.shape, full),
          pl.BlockSpec((1, 32), full),
          pl.BlockSpec(W3.shape, full),
          pl.BlockSpec((1, 16), full),
          pl.BlockSpec(W4.shape, full),
          pl.BlockSpec((1, 3), full),
      ],
      out_specs=pl.BlockSpec((_BM, 3), lambda i: (i, 0)),
      out_shape=jax.ShapeDtypeStruct((BATCH, 3), jnp.float32),
  )(g2, g3, p2.reshape(BATCH, 1), p3.reshape(BATCH, 1), w1a, w1b,
    b1.reshape(1, 32), W2, b2.reshape(1, 32), W3, b3.reshape(1, 16), W4,
    b4.reshape(1, 3))


def kernel(X_2, X_3, E2, E3, W1, b1, W2, b2, W3, b3, W4, b4):
  x2 = X_2.astype(jnp.int32)
  x3 = X_3.astype(jnp.int32)
  h2 = _CB * ((E2.shape[0] // 2 + _CB - 1) // _CB)
  h3 = _CB * ((E3.shape[0] // 2 + _CB - 1) // _CB)
  e2v = _repack(jnp.transpose(E2), h2)
  e3v = _repack(jnp.transpose(E3), h3)
  s2 = (x2 >= h2).astype(jnp.int32)
  s3 = (x3 >= h3).astype(jnp.int32)
  g2, g3 = _sc_gather(x2 - s2 * h2, x3 - s3 * h3, e2v, e3v)
  return _mlp(g2, g3, s2, s3, W1, b1, W2, b2, W3, b3, W4, b4)


# speedup vs baseline: 1.3712x; 1.0285x over previous
import jax
import jax.numpy as jnp
from jax import lax
from jax.experimental import pallas as pl
from jax.experimental.pallas import tpu as pltpu
from jax.experimental.pallas import tpu_sc as plsc

EDIM = 64
BATCH = 16384

_NC = 2
_NS = 16
_NW = _NC * _NS
_BPW = BATCH // _NW
_CHUNK = 128
_NCHUNK = _BPW // _CHUNK

_CB = 16384


def _repack_body(ta_ref, tb_ref, out_ref):
  eye = (lax.broadcasted_iota(jnp.int32, (EDIM, EDIM), 0) ==
         lax.broadcasted_iota(jnp.int32, (EDIM, EDIM), 1)).astype(jnp.float32)
  dn = (((0,), (0,)), ((), ()))
  out_ref[:, :EDIM] = lax.dot_general(
      ta_ref[...], eye, dn, preferred_element_type=jnp.float32)
  out_ref[:, EDIM:] = lax.dot_general(
      tb_ref[...], eye, dn, preferred_element_type=jnp.float32)


def _repack(table_t, half):
  n = table_t.shape[1]
  steps = half // _CB
  last = (n + _CB - 1) // _CB - 1
  return pl.pallas_call(
      _repack_body,
      grid=(steps,),
      in_specs=[
          pl.BlockSpec((EDIM, _CB), lambda i: (0, i)),
          pl.BlockSpec((EDIM, _CB),
                       lambda i: (0, jnp.minimum(i + steps, last))),
      ],
      out_specs=pl.BlockSpec((_CB, 2 * EDIM), lambda i: (i, 0)),
      out_shape=jax.ShapeDtypeStruct((half, 2 * EDIM), jnp.float32),
  )(table_t, table_t)


def _sc_gather_body(x2_hbm, x3_hbm, e2_hbm, e3_hbm, out2_hbm, out3_hbm,
                    idx_v, rows_v, sem):
  wid = lax.axis_index("s") * _NC + lax.axis_index("c")
  base = wid * _BPW
  pltpu.sync_copy(x2_hbm.at[pl.ds(wid * _NCHUNK, _NCHUNK)], idx_v.at[0])
  pltpu.sync_copy(x3_hbm.at[pl.ds(wid * _NCHUNK, _NCHUNK)], idx_v.at[1])
  for t, (e_hbm, out_hbm) in enumerate(((e2_hbm, out2_hbm),
                                        (e3_hbm, out3_hbm))):
    copies = [
        pltpu.async_copy(e_hbm.at[idx_v.at[t].at[c]],
                         rows_v.at[pl.ds(c * _CHUNK, _CHUNK)], sem)
        for c in range(_NCHUNK)
    ]
    for cp in copies:
      cp.wait()
    pltpu.sync_copy(rows_v, out_hbm.at[pl.ds(base, _BPW)])


def _sc_gather(x2h, x3h, e2v, e3v):
  mesh = plsc.VectorSubcoreMesh(core_axis_name="c", subcore_axis_name="s")
  f = pl.kernel(
      _sc_gather_body,
      mesh=mesh,
      out_type=(
          jax.ShapeDtypeStruct((BATCH, 2 * EDIM), jnp.float32),
          jax.ShapeDtypeStruct((BATCH, 2 * EDIM), jnp.float32),
      ),
      scratch_types=[
          pltpu.VMEM((2, _NCHUNK, _CHUNK), jnp.int32),
          pltpu.VMEM((_BPW, 2 * EDIM), jnp.float32),
          pltpu.SemaphoreType.DMA,
      ],
  )
  return f(x2h.reshape(_NW * _NCHUNK, _CHUNK),
           x3h.reshape(_NW * _NCHUNK, _CHUNK), e2v, e3v)


_BM = 2048


def _mlp_body(g2_ref, g3_ref, p2_ref, p3_ref, w1a_ref, w1b_ref, b1_ref,
              w2_ref, b2_ref, w3_ref, b3_ref, w4_ref, b4_ref, out_ref):
  g2 = g2_ref[...]
  g3 = g3_ref[...]
  emb2 = jnp.where(p2_ref[...] > 0, g2[:, EDIM:], g2[:, :EDIM])
  emb3 = jnp.where(p3_ref[...] > 0, g3[:, EDIM:], g3[:, :EDIM])
  h = jnp.dot(emb2, w1a_ref[...], preferred_element_type=jnp.float32)
  h = h + jnp.dot(emb3, w1b_ref[...], preferred_element_type=jnp.float32)
  h = jnp.maximum(h + b1_ref[...], 0.0)
  h = jnp.maximum(
      jnp.dot(h, w2_ref[...], preferred_element_type=jnp.float32) + b2_ref[...],
      0.0)
  h = jnp.maximum(
      jnp.dot(h, w3_ref[...], preferred_element_type=jnp.float32) + b3_ref[...],
      0.0)
  out_ref[...] = (
      jnp.dot(h, w4_ref[...], preferred_element_type=jnp.float32) + b4_ref[...])


def _mlp(g2, g3, p2, p3, W1, b1, W2, b2, W3, b3, W4, b4):
  w1a = W1[:EDIM]
  w1b = W1[EDIM:]
  full = lambda i: (0, 0)
  return pl.pallas_call(
      _mlp_body,
      grid=(BATCH // _BM,),
      in_specs=[
          pl.BlockSpec((_BM, 2 * EDIM), lambda i: (i, 0)),
          pl.BlockSpec((_BM, 2 * EDIM), lambda i: (i, 0)),
          pl.BlockSpec((_BM, 1), lambda i: (i, 0)),
          pl.BlockSpec((_BM, 1), lambda i: (i, 0)),
          pl.BlockSpec(w1a.shape, full),
          pl.BlockSpec(w1b.shape, full),
          pl.BlockSpec((1, 32), full),
          pl.BlockSpec(W2.shape, full),
          pl.BlockSpec((1, 32), full),
          pl.BlockSpec(W3.shape, full),
          pl.BlockSpec((1, 16), full),
          pl.BlockSpec(W4.shape, full),
          pl.BlockSpec((1, 3), full),
      ],
      out_specs=pl.BlockSpec((_BM, 3), lambda i: (i, 0)),
      out_shape=jax.ShapeDtypeStruct((BATCH, 3), jnp.float32),
  )(g2, g3, p2.reshape(BATCH, 1), p3.reshape(BATCH, 1), w1a, w1b,
    b1.reshape(1, 32), W2, b2.reshape(1, 32), W3, b3.reshape(1, 16), W4,
    b4.reshape(1, 3))


def kernel(X_2, X_3, E2, E3, W1, b1, W2, b2, W3, b3, W4, b4):
  x2 = X_2.astype(jnp.int32)
  x3 = X_3.astype(jnp.int32)
  h2 = _CB * ((E2.shape[0] // 2 + _CB - 1) // _CB)
  h3 = _CB * ((E3.shape[0] // 2 + _CB - 1) // _CB)
  e2v = _repack(jnp.transpose(E2), h2)
  e3v = _repack(jnp.transpose(E3), h3)
  s2 = (x2 >= h2).astype(jnp.int32)
  s3 = (x3 >= h3).astype(jnp.int32)
  g2, g3 = _sc_gather(x2 - s2 * h2, x3 - s3 * h3, e2v, e3v)
  return _mlp(g2, g3, s2, s3, W1, b1, W2, b2, W3, b3, W4, b4)
